# baseline (speedup 1.0000x reference)
import jax
import jax.numpy as jnp
from jax import lax
from jax.experimental import pallas as pl
from jax.experimental.pallas import tpu as pltpu
from jax.experimental.pallas import tpu_sc as plsc

BATCH = 16384
EMBED_DIM = 32
VOCAB = 1000000
ROWS_PER_128 = 128 // EMBED_DIM
PACKED_ROWS = VOCAB // ROWS_PER_128
FULL_COLS = VOCAB // 128
TAIL_START = FULL_COLS * 128
TAIL_PACKED = PACKED_ROWS - TAIL_START // ROWS_PER_128
NUM_CORES = 2
NUM_SUBCORES = 16
NUM_WORKERS = NUM_CORES * NUM_SUBCORES
BPW = BATCH // NUM_WORKERS
CHUNK = 128
NCHUNK = BPW // CHUNK
LANES = 16
GPC = CHUNK // LANES
KMAX = FULL_COLS // NUM_WORKERS + 1


def _tp_body(uembt_hbm, iembt_hbm, tail_u_hbm, tail_i_hbm, u128_hbm, i128_hbm,
             inu0, inu1, ini0, ini1, outu0, outu1, outi0, outi1, tailbuf,
             siu0, siu1, sii0, sii1, sou0, sou1, soi0, soi1):
  wid = lax.axis_index("s") * NUM_CORES + lax.axis_index("c")
  inu, ini = (inu0, inu1), (ini0, ini1)
  outu, outi = (outu0, outu1), (outi0, outi1)
  siu, sii = (siu0, siu1), (sii0, sii1)
  sou, soi = (sou0, sou1), (soi0, soi1)
  iota16 = lax.iota(jnp.int32, 16)

  def col_of(k):
    return k * NUM_WORKERS + wid

  def valid(k):
    return col_of(k) < FULL_COLS

  def fire_in(k, p):
    @pl.when(valid(k))
    def _():
      off = pl.multiple_of(col_of(k) * 128, 128)
      pltpu.async_copy(uembt_hbm.at[:, pl.ds(off, 128)], inu[p], siu[p])
      pltpu.async_copy(iembt_hbm.at[:, pl.ds(off, 128)], ini[p], sii[p])

  def wait_in(k, p):
    @pl.when(valid(k))
    def _():
      pltpu.make_async_copy(uembt_hbm.at[:, pl.ds(0, 128)], inu[p],
                            siu[p]).wait()
      pltpu.make_async_copy(iembt_hbm.at[:, pl.ds(0, 128)], ini[p],
                            sii[p]).wait()

  def wait_out(k, p):
    @pl.when(valid(k))
    def _():
      pltpu.make_async_copy(outu[p], u128_hbm.at[pl.ds(0, 32), :],
                            sou[p]).wait()
      pltpu.make_async_copy(outi[p], i128_hbm.at[pl.ds(0, 32), :],
                            soi[p]).wait()

  def transpose_and_flush(k, p):
    @pl.when(valid(k))
    def _():
      for src, dst in ((inu[p], outu[p]), (ini[p], outi[p])):
        for rp in range(32):
          for l0 in range(0, 128, LANES):
            dvec = (l0 % 32) + iota16
            cvec = jnp.full((LANES,), rp * 4 + l0 // 32, jnp.int32)
            dst[rp, pl.ds(l0, LANES)] = plsc.load_gather(src, [dvec, cvec])
      row0 = pl.multiple_of(col_of(k) * 32, 8)
      pltpu.async_copy(outu[p], u128_hbm.at[pl.ds(row0, 32), :], sou[p])
      pltpu.async_copy(outi[p], i128_hbm.at[pl.ds(row0, 32), :], soi[p])

  @pl.when(wid == 0)
  def _():
    pltpu.sync_copy(tail_u_hbm, tailbuf)
    pltpu.sync_copy(tailbuf, u128_hbm.at[pl.ds(PACKED_ROWS - TAIL_PACKED,
                                               TAIL_PACKED), :])
  @pl.when(wid == 1)
  def _():
    pltpu.sync_copy(tail_i_hbm, tailbuf)
    pltpu.sync_copy(tailbuf, i128_hbm.at[pl.ds(PACKED_ROWS - TAIL_PACKED,
                                               TAIL_PACKED), :])

  fire_in(0, 0)
  fire_in(1, 1)

  def step(kk, carry):
    for p in range(2):
      k = 2 * kk + p
      wait_in(k, p)
      @pl.when(k >= 2)
      def _():
        wait_out(k - 2, p)
      transpose_and_flush(k, p)
      fire_in(k + 2, p)
    return carry

  lax.fori_loop(0, (KMAX + 1) // 2, step, 0)

  for k in (KMAX - 1, KMAX):
    wait_out(k, k % 2)


def _mf_body(uidx_hbm, iidx_hbm, uemb_hbm, iemb_hbm, ulin_hbm, ilin_hbm,
             out_hbm, uidx_v, iidx_v, urow_idx, irow_idx,
             urows_a, irows_a, urows_b, irows_b,
             ubias_v, ibias_v, out_v, bias_sem, sem_a, sem_b):
  wid = lax.axis_index("s") * NUM_CORES + lax.axis_index("c")

  pltpu.sync_copy(uidx_hbm.at[pl.ds(wid * BPW, BPW)], uidx_v)
  pltpu.sync_copy(iidx_hbm.at[pl.ds(wid * BPW, BPW)], iidx_v)

  iota16 = lax.iota(jnp.int32, 16)
  ubufs, ibufs, sems = (urows_a, urows_b), (irows_a, irows_b), (sem_a, sem_b)

  def prep_rows(c):
    for g in range(GPC):
      sl = pl.ds(c * CHUNK + g * LANES, LANES)
      urow_idx[sl] = uidx_v[sl] // ROWS_PER_128
      irow_idx[sl] = iidx_v[sl] // ROWS_PER_128

  def fire(c):
    p = c % 2
    rows = pl.ds(c * CHUNK, CHUNK)
    return (pltpu.async_copy(uemb_hbm.at[urow_idx.at[rows]], ubufs[p], sems[p]),
            pltpu.async_copy(iemb_hbm.at[irow_idx.at[rows]], ibufs[p], sems[p]))

  def compute(c):
    p = c % 2
    for g in range(GPC):
      sl = pl.ds(c * CHUNK + g * LANES, LANES)
      slot = g * LANES + iota16
      uoff = (uidx_v[sl] & (ROWS_PER_128 - 1)) * EMBED_DIM
      ioff = (iidx_v[sl] & (ROWS_PER_128 - 1)) * EMBED_DIM
      acc = ubias_v[sl] + ibias_v[sl]
      for d in range(EMBED_DIM):
        u = plsc.load_gather(ubufs[p], [slot, uoff + d])
        it = plsc.load_gather(ibufs[p], [slot, ioff + d])
        acc = acc + u * it
      out_v[sl] = 1.0 / (1.0 + jnp.exp(-acc))

  bias_copies = []
  for c in range(NCHUNK):
    rows = pl.ds(c * CHUNK, CHUNK)
    bias_copies.append(pltpu.async_copy(
        ulin_hbm.at[uidx_v.at[rows]], ubias_v.at[rows], bias_sem))
    bias_copies.append(pltpu.async_copy(
        ilin_hbm.at[iidx_v.at[rows]], ibias_v.at[rows], bias_sem))

  prep_rows(0)
  inflight = [fire(0)]
  prep_rows(1)
  inflight.append(fire(1))
  for cp in bias_copies:
    cp.wait()

  for c in range(NCHUNK):
    for cp in inflight[c]:
      cp.wait()
    compute(c)
    if c + 2 < NCHUNK:
      prep_rows(c + 2)
      inflight.append(fire(c + 2))

  pltpu.sync_copy(out_v, out_hbm.at[pl.ds(wid * BPW, BPW)])


@jax.jit
def _mf_call(uidx, iidx, uembt, iembt, tail_u, tail_i, ulin_flat, ilin_flat):
  mesh = plsc.VectorSubcoreMesh(core_axis_name="c", subcore_axis_name="s")

  tp = pl.kernel(
      _tp_body,
      out_type=(jax.ShapeDtypeStruct((PACKED_ROWS, 128), jnp.float32),
                jax.ShapeDtypeStruct((PACKED_ROWS, 128), jnp.float32)),
      mesh=mesh,
      scratch_types=(
          [pltpu.VMEM((32, 128), jnp.float32)] * 8 +
          [pltpu.VMEM((TAIL_PACKED, 128), jnp.float32)] +
          [pltpu.SemaphoreType.DMA] * 8
      ),
      compiler_params=pltpu.CompilerParams(needs_layout_passes=False),
  )
  u128, i128 = tp(uembt, iembt, tail_u, tail_i)

  fn = pl.kernel(
      _mf_body,
      out_type=jax.ShapeDtypeStruct((BATCH,), jnp.float32),
      mesh=mesh,
      scratch_types=[
          pltpu.VMEM((BPW,), jnp.int32),
          pltpu.VMEM((BPW,), jnp.int32),
          pltpu.VMEM((BPW,), jnp.int32),
          pltpu.VMEM((BPW,), jnp.int32),
          pltpu.VMEM((CHUNK, 128), jnp.float32),
          pltpu.VMEM((CHUNK, 128), jnp.float32),
          pltpu.VMEM((CHUNK, 128), jnp.float32),
          pltpu.VMEM((CHUNK, 128), jnp.float32),
          pltpu.VMEM((BPW,), jnp.float32),
          pltpu.VMEM((BPW,), jnp.float32),
          pltpu.VMEM((BPW,), jnp.float32),
          pltpu.SemaphoreType.DMA,
          pltpu.SemaphoreType.DMA,
          pltpu.SemaphoreType.DMA,
      ],
      compiler_params=pltpu.CompilerParams(needs_layout_passes=False),
  )
  return fn(uidx, iidx, u128, i128, ulin_flat, ilin_flat)


def kernel(user_tensor, item_tensor, user_embed, item_embed, user_lin,
           item_lin):
  tail_u = user_embed[TAIL_START:].reshape(TAIL_PACKED, 128)
  tail_i = item_embed[TAIL_START:].reshape(TAIL_PACKED, 128)
  return _mf_call(user_tensor.astype(jnp.int32),
                  item_tensor.astype(jnp.int32),
                  user_embed.T, item_embed.T, tail_u, tail_i,
                  user_lin.reshape(-1), item_lin.reshape(-1))
